# baseline (speedup 1.0000x reference)
import functools

import jax
import jax.numpy as jnp
from jax import lax
from jax.experimental import pallas as pl
from jax.experimental.pallas import tpu as pltpu
from jax.experimental.pallas import tpu_sc as plsc

_CR = 16
_L = 16
_NF = 512


@functools.partial(jax.jit, static_argnames=("n", "n_seg"))
def _sc_energy(x, seg_ids, W, b, n, n_seg):
    NW = 32
    rows_per_w = ((n + NW * _L - 1) // (NW * _L)) * _L
    full_ch = rows_per_w // _CR
    last_rows = n - (NW - 1) * rows_per_w
    assert last_rows % _CR == 0 and last_rows > 0
    last_ch = last_rows // _CR
    npairs = (full_ch + 1) // 2
    acc_len = ((n_seg + _L - 1) // _L) * _L
    ncols = acc_len // _L
    reps = (ncols + 15) // 16

    mesh = plsc.VectorSubcoreMesh(core_axis_name="c", subcore_axis_name="s")

    @functools.partial(
        pl.kernel,
        mesh=mesh,
        out_type=jax.ShapeDtypeStruct((2, n_seg), jnp.float32),
        compiler_params=pltpu.CompilerParams(needs_layout_passes=False),
        scratch_types=[
            pltpu.VMEM((_CR, _NF), jnp.float32),
            pltpu.VMEM((_CR, _NF), jnp.float32),
            pltpu.VMEM((_CR,), jnp.int32),
            pltpu.VMEM((_CR,), jnp.int32),
            pltpu.VMEM((_NF,), jnp.float32),
            pltpu.VMEM((_L,), jnp.float32),
            pltpu.VMEM((acc_len,), jnp.float32),
            pltpu.VMEM((16 * _L,), jnp.float32),
            pltpu.VMEM_SHARED((acc_len * 16,), jnp.float32),
            pltpu.SemaphoreType.DMA,
            pltpu.SemaphoreType.DMA,
        ],
    )
    def energy(x_hbm, seg_hbm, w_hbm, b_hbm, out_hbm,
               xb0, xb1, ib0, ib1, w_v, b_v, acc, colbuf, shared,
               sem0, sem1):
        c = lax.axis_index("c")
        s = lax.axis_index("s")
        wid = c * 16 + s
        base = wid * rows_per_w
        nch = jnp.where(wid == NW - 1, last_ch, full_ch)

        pltpu.sync_copy(w_hbm, w_v)
        pltpu.sync_copy(b_hbm, b_v)
        bvec = b_v[...]
        lane = lax.iota(jnp.int32, _L)
        masks = [lane == r for r in range(_L)]
        zeros = jnp.zeros((_L,), jnp.float32)
        for j in range(ncols):
            acc[pl.ds(j * _L, _L)] = zeros

        def cp(i, xb, ib, sem):
            row0 = base + i * _CR
            return (
                pltpu.make_async_copy(x_hbm.at[pl.ds(row0, _CR)], xb, sem),
                pltpu.make_async_copy(seg_hbm.at[pl.ds(row0, _CR)], ib, sem),
            )

        def start(i, xb, ib, sem):
            a, bcp = cp(i, xb, ib, sem)
            a.start()
            bcp.start()

        def wait(i, xb, ib, sem):
            a, bcp = cp(i, xb, ib, sem)
            a.wait()
            bcp.wait()

        def process(xb, ib):
            for g in range(_CR // _L):
                yv = zeros
                for sub in range(4):
                    r0 = g * _L + sub * 4
                    a = [None] * 4
                    for j in range(_NF // _L):
                        w = w_v[pl.ds(j * _L, _L)]
                        for q in range(4):
                            p = xb[r0 + q, pl.ds(j * _L, _L)] * w
                            a[q] = p if a[q] is None else a[q] + p
                    for q in range(4):
                        ys = lax.broadcast_in_dim(jnp.sum(a[q]), (_L,), ())
                        yv = jnp.where(masks[sub * 4 + q], ys, yv)
                yv = yv + bvec
                iv = ib[pl.ds(g * _L, _L)]
                plsc.addupdate_scatter(acc, [iv], yv)

        start(0, xb0, ib0, sem0)

        def body(p, carry):
            i1 = 2 * p + 1
            i2 = 2 * p + 2

            @pl.when(i1 < nch)
            def _():
                start(i1, xb1, ib1, sem1)

            @pl.when(2 * p < nch)
            def _():
                wait(2 * p, xb0, ib0, sem0)
                process(xb0, ib0)

            @pl.when(i2 < nch)
            def _():
                start(i2, xb0, ib0, sem0)

            @pl.when(i1 < nch)
            def _():
                wait(i1, xb1, ib1, sem1)
                process(xb1, ib1)

            return carry

        lax.fori_loop(0, npairs, body, 0)

        for j in range(ncols):
            pltpu.sync_copy(acc.at[pl.ds(j * _L, _L)],
                            shared.at[pl.ds((j * 16 + s) * _L, _L)])
        plsc.subcore_barrier()

        for rep in range(reps):
            col = s + rep * 16

            @pl.when(col < ncols)
            def _():
                pltpu.sync_copy(shared.at[pl.ds(col * 16 * _L, 16 * _L)],
                                colbuf)
                tot = zeros
                for k in range(16):
                    tot = tot + colbuf[pl.ds(k * _L, _L)]
                acc[pl.ds(0, _L)] = tot
                pltpu.sync_copy(acc.at[pl.ds(0, _L)],
                                out_hbm.at[c].at[pl.ds(col * _L, _L)])

    return energy(x, seg_ids, W, b)


def _combine_body(a_ref, o_ref):
    o_ref[...] = (a_ref[0] + a_ref[1]).reshape(-1, 1)


def _combine(partials, n_seg):
    return pl.pallas_call(
        _combine_body,
        out_shape=jax.ShapeDtypeStruct((n_seg, 1), jnp.float32),
    )(partials)


def kernel(x, atomic_subsystem_counts, W, b):
    n, _ = x.shape
    n_seg = atomic_subsystem_counts.shape[0]
    counts = atomic_subsystem_counts.astype(jnp.int32)
    seg_ids = jnp.repeat(
        jnp.arange(n_seg, dtype=jnp.int32), counts, total_repeat_length=n)
    w_flat = W.reshape(_NF)
    b16 = jnp.broadcast_to(b, (_L,))
    partials = _sc_energy(x, seg_ids, w_flat, b16, n=n, n_seg=n_seg)
    return _combine(partials, n_seg)


# speedup vs baseline: 1.1236x; 1.1236x over previous
import functools

import jax
import jax.numpy as jnp
from jax import lax
from jax.experimental import pallas as pl
from jax.experimental.pallas import tpu as pltpu
from jax.experimental.pallas import tpu_sc as plsc

_CR = 112
_L = 16
_NF = 512


@functools.partial(jax.jit, static_argnames=("n", "n_seg"))
def _sc_energy(x, seg_ids, W, b, n, n_seg):
    NW = 32
    rows_per_w = ((n + NW * _L - 1) // (NW * _L)) * _L
    full_ch = rows_per_w // _CR
    last_rows = n - (NW - 1) * rows_per_w
    assert last_rows % _CR == 0 and last_rows > 0
    last_ch = last_rows // _CR
    npairs = (full_ch + 1) // 2
    acc_len = ((n_seg + _L - 1) // _L) * _L
    ncols = acc_len // _L
    reps = (ncols + 15) // 16

    mesh = plsc.VectorSubcoreMesh(core_axis_name="c", subcore_axis_name="s")

    @functools.partial(
        pl.kernel,
        mesh=mesh,
        out_type=jax.ShapeDtypeStruct((2, n_seg), jnp.float32),
        compiler_params=pltpu.CompilerParams(needs_layout_passes=False),
        scratch_types=[
            pltpu.VMEM((_CR, _NF), jnp.float32),
            pltpu.VMEM((_CR, _NF), jnp.float32),
            pltpu.VMEM((_CR,), jnp.int32),
            pltpu.VMEM((_CR,), jnp.int32),
            pltpu.VMEM((_NF,), jnp.float32),
            pltpu.VMEM((_L,), jnp.float32),
            pltpu.VMEM((acc_len,), jnp.float32),
            pltpu.VMEM((16 * _L,), jnp.float32),
            pltpu.VMEM_SHARED((acc_len * 16,), jnp.float32),
            pltpu.SemaphoreType.DMA,
            pltpu.SemaphoreType.DMA,
        ],
    )
    def energy(x_hbm, seg_hbm, w_hbm, b_hbm, out_hbm,
               xb0, xb1, ib0, ib1, w_v, b_v, acc, colbuf, shared,
               sem0, sem1):
        c = lax.axis_index("c")
        s = lax.axis_index("s")
        wid = c * 16 + s
        base = wid * rows_per_w
        nch = jnp.where(wid == NW - 1, last_ch, full_ch)

        pltpu.sync_copy(w_hbm, w_v)
        pltpu.sync_copy(b_hbm, b_v)
        bvec = b_v[...]
        lane = lax.iota(jnp.int32, _L)
        masks = [lane == r for r in range(_L)]
        zeros = jnp.zeros((_L,), jnp.float32)
        for j in range(ncols):
            acc[pl.ds(j * _L, _L)] = zeros

        def cp(i, xb, ib, sem):
            row0 = base + i * _CR
            return (
                pltpu.make_async_copy(x_hbm.at[pl.ds(row0, _CR)], xb, sem),
                pltpu.make_async_copy(seg_hbm.at[pl.ds(row0, _CR)], ib, sem),
            )

        def start(i, xb, ib, sem):
            a, bcp = cp(i, xb, ib, sem)
            a.start()
            bcp.start()

        def wait(i, xb, ib, sem):
            a, bcp = cp(i, xb, ib, sem)
            a.wait()
            bcp.wait()

        def process(xb, ib):
            def group(g, carry):
                yv = zeros
                for sub in range(4):
                    r0 = g * _L + sub * 4
                    a = [None] * 4
                    for j in range(_NF // _L):
                        w = w_v[pl.ds(j * _L, _L)]
                        for q in range(4):
                            p = xb[r0 + q, pl.ds(j * _L, _L)] * w
                            a[q] = p if a[q] is None else a[q] + p
                    for q in range(4):
                        ys = lax.broadcast_in_dim(jnp.sum(a[q]), (_L,), ())
                        yv = jnp.where(masks[sub * 4 + q], ys, yv)
                yv = yv + bvec
                iv = ib[pl.ds(pl.multiple_of(g * _L, _L), _L)]
                plsc.addupdate_scatter(acc, [iv], yv)
                return carry

            lax.fori_loop(0, _CR // _L, group, 0)

        start(0, xb0, ib0, sem0)

        def body(p, carry):
            i1 = 2 * p + 1
            i2 = 2 * p + 2

            @pl.when(i1 < nch)
            def _():
                start(i1, xb1, ib1, sem1)

            @pl.when(2 * p < nch)
            def _():
                wait(2 * p, xb0, ib0, sem0)
                process(xb0, ib0)

            @pl.when(i2 < nch)
            def _():
                start(i2, xb0, ib0, sem0)

            @pl.when(i1 < nch)
            def _():
                wait(i1, xb1, ib1, sem1)
                process(xb1, ib1)

            return carry

        lax.fori_loop(0, npairs, body, 0)

        for j in range(ncols):
            pltpu.sync_copy(acc.at[pl.ds(j * _L, _L)],
                            shared.at[pl.ds((j * 16 + s) * _L, _L)])
        plsc.subcore_barrier()

        for rep in range(reps):
            col = s + rep * 16

            @pl.when(col < ncols)
            def _():
                pltpu.sync_copy(shared.at[pl.ds(col * 16 * _L, 16 * _L)],
                                colbuf)
                tot = zeros
                for k in range(16):
                    tot = tot + colbuf[pl.ds(k * _L, _L)]
                acc[pl.ds(0, _L)] = tot
                pltpu.sync_copy(acc.at[pl.ds(0, _L)],
                                out_hbm.at[c].at[pl.ds(col * _L, _L)])

    return energy(x, seg_ids, W, b)


def _combine_body(a_ref, o_ref):
    o_ref[...] = (a_ref[0] + a_ref[1]).reshape(-1, 1)


def _combine(partials, n_seg):
    return pl.pallas_call(
        _combine_body,
        out_shape=jax.ShapeDtypeStruct((n_seg, 1), jnp.float32),
    )(partials)


def kernel(x, atomic_subsystem_counts, W, b):
    n, _ = x.shape
    n_seg = atomic_subsystem_counts.shape[0]
    counts = atomic_subsystem_counts.astype(jnp.int32)
    seg_ids = jnp.repeat(
        jnp.arange(n_seg, dtype=jnp.int32), counts, total_repeat_length=n)
    w_flat = W.reshape(_NF)
    b16 = jnp.broadcast_to(b, (_L,))
    partials = _sc_energy(x, seg_ids, w_flat, b16, n=n, n_seg=n_seg)
    return _combine(partials, n_seg)


# speedup vs baseline: 1.2691x; 1.1295x over previous
import functools

import jax
import jax.numpy as jnp
from jax import lax
from jax.experimental import pallas as pl
from jax.experimental.pallas import tpu as pltpu
from jax.experimental.pallas import tpu_sc as plsc

_CR = 112
_L = 16
_NF = 512


@functools.partial(jax.jit, static_argnames=("n", "n_seg"))
def _sc_energy(x, seg_ids, W, b, n, n_seg):
    NW = 32
    rows_per_w = ((n + NW * _L - 1) // (NW * _L)) * _L
    full_ch = rows_per_w // _CR
    last_rows = n - (NW - 1) * rows_per_w
    assert last_rows % _CR == 0 and last_rows > 0
    last_ch = last_rows // _CR
    npairs = (full_ch + 1) // 2
    acc_len = ((n_seg + _L - 1) // _L) * _L
    ncols = acc_len // _L
    reps = (ncols + 15) // 16

    mesh = plsc.VectorSubcoreMesh(core_axis_name="c", subcore_axis_name="s")

    @functools.partial(
        pl.kernel,
        mesh=mesh,
        out_type=jax.ShapeDtypeStruct((2, n_seg), jnp.float32),
        compiler_params=pltpu.CompilerParams(needs_layout_passes=False),
        scratch_types=[
            pltpu.VMEM((_CR, _NF), jnp.float32),
            pltpu.VMEM((_CR, _NF), jnp.float32),
            pltpu.VMEM((_CR,), jnp.int32),
            pltpu.VMEM((_CR,), jnp.int32),
            pltpu.VMEM((_NF,), jnp.float32),
            pltpu.VMEM((_L,), jnp.float32),
            pltpu.VMEM((_L * _L,), jnp.float32),
            pltpu.VMEM((acc_len,), jnp.float32),
            pltpu.VMEM((16 * _L,), jnp.float32),
            pltpu.VMEM_SHARED((acc_len * 16,), jnp.float32),
            pltpu.SemaphoreType.DMA,
            pltpu.SemaphoreType.DMA,
        ],
    )
    def energy(x_hbm, seg_hbm, w_hbm, b_hbm, out_hbm,
               xb0, xb1, ib0, ib1, w_v, b_v, tbuf, acc, colbuf, shared,
               sem0, sem1):
        c = lax.axis_index("c")
        s = lax.axis_index("s")
        wid = c * 16 + s
        base = wid * rows_per_w
        nch = jnp.where(wid == NW - 1, last_ch, full_ch)

        pltpu.sync_copy(w_hbm, w_v)
        pltpu.sync_copy(b_hbm, b_v)
        bvec = b_v[...]
        lane = lax.iota(jnp.int32, _L)
        masks = [lane == r for r in range(_L)]
        zeros = jnp.zeros((_L,), jnp.float32)
        for j in range(ncols):
            acc[pl.ds(j * _L, _L)] = zeros

        def cp(i, xb, ib, sem):
            row0 = base + i * _CR
            return (
                pltpu.make_async_copy(x_hbm.at[pl.ds(row0, _CR)], xb, sem),
                pltpu.make_async_copy(seg_hbm.at[pl.ds(row0, _CR)], ib, sem),
            )

        def start(i, xb, ib, sem):
            a, bcp = cp(i, xb, ib, sem)
            a.start()
            bcp.start()

        def wait(i, xb, ib, sem):
            a, bcp = cp(i, xb, ib, sem)
            a.wait()
            bcp.wait()

        lane16 = lane * _L
        gidx = [lane16 + col for col in range(_L)]

        def process(xb, ib):
            def group(g, carry):
                for sub in range(4):
                    r0 = g * _L + sub * 4
                    a = [None] * 4
                    for j in range(_NF // _L):
                        w = w_v[pl.ds(j * _L, _L)]
                        for q in range(4):
                            p = xb[r0 + q, pl.ds(j * _L, _L)] * w
                            a[q] = p if a[q] is None else a[q] + p
                    for q in range(4):
                        tbuf[pl.ds((sub * 4 + q) * _L, _L)] = a[q]
                y0 = None
                for col in range(_L):
                    v = plsc.load_gather(tbuf, [gidx[col]])
                    y0 = v if y0 is None else y0 + v
                yv = y0 + bvec
                iv = ib[pl.ds(pl.multiple_of(g * _L, _L), _L)]
                plsc.addupdate_scatter(acc, [iv], yv)
                return carry

            lax.fori_loop(0, _CR // _L, group, 0)

        start(0, xb0, ib0, sem0)

        def body(p, carry):
            i1 = 2 * p + 1
            i2 = 2 * p + 2

            @pl.when(i1 < nch)
            def _():
                start(i1, xb1, ib1, sem1)

            @pl.when(2 * p < nch)
            def _():
                wait(2 * p, xb0, ib0, sem0)
                process(xb0, ib0)

            @pl.when(i2 < nch)
            def _():
                start(i2, xb0, ib0, sem0)

            @pl.when(i1 < nch)
            def _():
                wait(i1, xb1, ib1, sem1)
                process(xb1, ib1)

            return carry

        lax.fori_loop(0, npairs, body, 0)

        for j in range(ncols):
            pltpu.sync_copy(acc.at[pl.ds(j * _L, _L)],
                            shared.at[pl.ds((j * 16 + s) * _L, _L)])
        plsc.subcore_barrier()

        for rep in range(reps):
            col = s + rep * 16

            @pl.when(col < ncols)
            def _():
                pltpu.sync_copy(shared.at[pl.ds(col * 16 * _L, 16 * _L)],
                                colbuf)
                tot = zeros
                for k in range(16):
                    tot = tot + colbuf[pl.ds(k * _L, _L)]
                acc[pl.ds(0, _L)] = tot
                pltpu.sync_copy(acc.at[pl.ds(0, _L)],
                                out_hbm.at[c].at[pl.ds(col * _L, _L)])

    return energy(x, seg_ids, W, b)


def _combine_body(a_ref, o_ref):
    o_ref[...] = (a_ref[0] + a_ref[1]).reshape(-1, 1)


def _combine(partials, n_seg):
    return pl.pallas_call(
        _combine_body,
        out_shape=jax.ShapeDtypeStruct((n_seg, 1), jnp.float32),
    )(partials)


def kernel(x, atomic_subsystem_counts, W, b):
    n, _ = x.shape
    n_seg = atomic_subsystem_counts.shape[0]
    counts = atomic_subsystem_counts.astype(jnp.int32)
    seg_ids = jnp.repeat(
        jnp.arange(n_seg, dtype=jnp.int32), counts, total_repeat_length=n)
    w_flat = W.reshape(_NF)
    b16 = jnp.broadcast_to(b, (_L,))
    partials = _sc_energy(x, seg_ids, w_flat, b16, n=n, n_seg=n_seg)
    return _combine(partials, n_seg)
